# Initial kernel scaffold; baseline (speedup 1.0000x reference)
import functools

import jax
import jax.numpy as jnp
from jax import lax
from jax.experimental import pallas as pl
from jax.experimental.pallas import tpu as pltpu
from jax.experimental.pallas import tpu_sc as plsc

V_SIZE = 1_000_000
E = 32
B_TOTAL = 4096 * 200

NUM_CORES = 2
NUM_SUBCORES = 16
NW = NUM_CORES * NUM_SUBCORES
BPW = B_TOTAL // NW
CHUNK = 1600
NCHUNK = BPW // CHUNK

_mesh = plsc.VectorSubcoreMesh(core_axis_name="c", subcore_axis_name="s")


@functools.partial(
    pl.kernel,
    mesh=_mesh,
    out_type=jax.ShapeDtypeStruct((B_TOTAL, E), jnp.float32),
    scratch_types=[
        pltpu.VMEM((CHUNK,), jnp.int32),
        pltpu.VMEM((CHUNK, E), jnp.float32),
        pltpu.SemaphoreType.DMA,
    ],
)
def _sc_gather(idx_hbm, table_hbm, out_hbm, idx_v, rows_v, sem):
    wid = lax.axis_index("s") * NUM_CORES + lax.axis_index("c")
    base = wid * BPW

    def body(i, carry):
        off = base + i * CHUNK
        pltpu.sync_copy(idx_hbm.at[pl.ds(off, CHUNK)], idx_v)
        pltpu.async_copy(table_hbm.at[idx_v], rows_v, sem).wait()
        pltpu.sync_copy(rows_v, out_hbm.at[pl.ds(off, CHUNK)])
        return carry

    lax.fori_loop(0, NCHUNK, body, 0)


def kernel(x, vocab):
    idx = x.reshape(-1).astype(jnp.int32)
    table = vocab.reshape(V_SIZE, E)
    out = _sc_gather(idx, table)
    return out.reshape(x.shape[0], x.shape[1], 1, E)


# speedup vs baseline: 1.4791x; 1.4791x over previous
import functools

import jax
import jax.numpy as jnp
from jax import lax
from jax.experimental import pallas as pl
from jax.experimental.pallas import tpu as pltpu
from jax.experimental.pallas import tpu_sc as plsc

V_SIZE = 1_000_000
E = 32
B_TOTAL = 4096 * 200

NUM_CORES = 2
NUM_SUBCORES = 16
NW = NUM_CORES * NUM_SUBCORES
BPW = B_TOTAL // NW
CHUNK = 1600
NCHUNK = BPW // CHUNK

_mesh = plsc.VectorSubcoreMesh(core_axis_name="c", subcore_axis_name="s")


@functools.partial(
    pl.kernel,
    mesh=_mesh,
    out_type=jax.ShapeDtypeStruct((B_TOTAL, E), jnp.float32),
    scratch_types=[
        pltpu.VMEM((CHUNK,), jnp.int32),
        pltpu.VMEM((CHUNK, E), jnp.float32),
        pltpu.SemaphoreType.DMA,
    ],
    compiler_params=pltpu.CompilerParams(use_tc_tiling_on_sc=False),
)
def _sc_gather(idx_hbm, table_hbm, out_hbm, idx_v, rows_v, sem):
    wid = lax.axis_index("s") * NUM_CORES + lax.axis_index("c")
    base = wid * BPW

    def body(i, carry):
        off = base + i * CHUNK
        pltpu.sync_copy(idx_hbm.at[pl.ds(off, CHUNK)], idx_v)
        pltpu.async_copy(table_hbm.at[idx_v], rows_v, sem).wait()
        pltpu.sync_copy(rows_v, out_hbm.at[pl.ds(off, CHUNK)])
        return carry

    lax.fori_loop(0, NCHUNK, body, 0)


def kernel(x, vocab):
    idx = x.reshape(-1).astype(jnp.int32)
    table = vocab.reshape(V_SIZE, E)
    out = _sc_gather(idx, table)
    return out.reshape(x.shape[0], x.shape[1], 1, E)


# speedup vs baseline: 1.5009x; 1.0147x over previous
import functools

import jax
import jax.numpy as jnp
from jax import lax
from jax.experimental import pallas as pl
from jax.experimental.pallas import tpu as pltpu
from jax.experimental.pallas import tpu_sc as plsc

V_SIZE = 1_000_000
E = 32
B_TOTAL = 4096 * 200

NUM_CORES = 2
NUM_SUBCORES = 16
NW = NUM_CORES * NUM_SUBCORES
BPW = B_TOTAL // NW
CHUNK = 640
NBUF = 4
NCHUNK = BPW // CHUNK
ROUNDS = NCHUNK // NBUF

_mesh = plsc.VectorSubcoreMesh(core_axis_name="c", subcore_axis_name="s")


@functools.partial(
    pl.kernel,
    mesh=_mesh,
    out_type=jax.ShapeDtypeStruct((B_TOTAL, E), jnp.float32),
    scratch_types=[
        pltpu.VMEM((BPW,), jnp.int32),
        pltpu.VMEM((NBUF, CHUNK, E), jnp.float32),
        pltpu.SemaphoreType.DMA,
        pltpu.SemaphoreType.DMA,
        pltpu.SemaphoreType.DMA,
        pltpu.SemaphoreType.DMA,
        pltpu.SemaphoreType.DMA,
        pltpu.SemaphoreType.DMA,
        pltpu.SemaphoreType.DMA,
        pltpu.SemaphoreType.DMA,
    ],
    compiler_params=pltpu.CompilerParams(use_tc_tiling_on_sc=False),
)
def _sc_gather(idx_hbm, table_hbm, out_hbm, idx_v, rows_v,
               g0, g1, g2, g3, w0, w1, w2, w3):
    gsem = (g0, g1, g2, g3)
    wsem = (w0, w1, w2, w3)
    wid = lax.axis_index("s") * NUM_CORES + lax.axis_index("c")
    base = wid * BPW

    pltpu.sync_copy(idx_hbm.at[pl.ds(base, BPW)], idx_v)

    def start_gather(chunk, buf):
        idx_slice = idx_v.at[pl.ds(chunk * CHUNK, CHUNK)]
        pltpu.async_copy(table_hbm.at[idx_slice], rows_v.at[buf], gsem[buf])

    for b in range(NBUF - 1):
        start_gather(b, b)

    @pl.loop(0, ROUNDS)
    def _round(r):
        for b in range(NBUF):
            c = r * NBUF + b
            bg = (b + NBUF - 1) % NBUF
            if b == 0:
                @pl.when(r > 0)
                def _():
                    pltpu.make_async_copy(
                        rows_v.at[bg], out_hbm.at[pl.ds(0, CHUNK)],
                        wsem[bg]).wait()
                start_gather(c + NBUF - 1, bg)
            else:
                pltpu.make_async_copy(
                    rows_v.at[bg], out_hbm.at[pl.ds(0, CHUNK)],
                    wsem[bg]).wait()

                @pl.when(r < ROUNDS - 1)
                def _():
                    start_gather(c + NBUF - 1, bg)
            pltpu.make_async_copy(
                table_hbm.at[idx_v.at[pl.ds(0, CHUNK)]], rows_v.at[b],
                gsem[b]).wait()
            pltpu.async_copy(
                rows_v.at[b], out_hbm.at[pl.ds(base + c * CHUNK, CHUNK)],
                wsem[b])

    pltpu.make_async_copy(
        rows_v.at[NBUF - 1], out_hbm.at[pl.ds(0, CHUNK)],
        wsem[NBUF - 1]).wait()


def kernel(x, vocab):
    idx = x.reshape(-1).astype(jnp.int32)
    table = vocab.reshape(V_SIZE, E)
    out = _sc_gather(idx, table)
    return out.reshape(x.shape[0], x.shape[1], 1, E)
